# baseline (speedup 1.0000x reference)
import jax
import jax.numpy as jnp
from jax.experimental import pallas as pl
from jax.experimental.pallas import tpu as pltpu

S = 2048
D = 2048
H = 16
KVH = 4
HD = 128
BLK = 64
NB = S // BLK
GH = 128
GROUP = H // KVH
RB = 256
NRB = S // RB
TK = 256
SCALE = HD ** -0.5
LOG2E = 1.4426950408889634
QSCALE = SCALE * LOG2E
THRESH = 1.0 / NB
NEG = -1e30


def _rot(x):
    h = x.shape[-1] // 2
    return jnp.concatenate([-x[..., h:], x[..., :h]], axis=-1)


def _proj_kernel(x_ref, w_ref, b_ref, cos_ref, sin_ref,
                 q_ref, k_ref, v_ref, qp_ref, kp_ref):
    x = x_ref[...]
    qkv = jnp.dot(x, w_ref[...], preferred_element_type=jnp.float32) + b_ref[...]
    c = cos_ref[...]
    s = sin_ref[...]
    for h in range(H):
        qh = qkv[:, h * HD:(h + 1) * HD]
        q4 = qh.reshape(RB // BLK, BLK, HD)
        qp_ref[:, h, :HD] = jnp.mean(q4, axis=1)
        qp_ref[:, h, HD:] = jnp.max(q4, axis=1)
        q_ref[h] = ((qh * c + _rot(qh) * s) * QSCALE).astype(jnp.bfloat16)
    for g in range(KVH):
        kh = qkv[:, H * HD + g * HD: H * HD + (g + 1) * HD]
        k4 = kh.reshape(RB // BLK, BLK, HD)
        kp_ref[:, g, :HD] = jnp.mean(k4, axis=1)
        kp_ref[:, g, HD:] = jnp.max(k4, axis=1)
        k_ref[g] = (kh * c + _rot(kh) * s).astype(jnp.bfloat16)
        v_ref[g] = qkv[:, (H + KVH) * HD + g * HD:
                       (H + KVH) * HD + (g + 1) * HD].astype(jnp.bfloat16)


def _gate_kernel(qp_ref, kp_ref, wqg_ref, wkg_ref, bc_ref, bs_ref, e_ref,
                 addm_ref):
    qg = jnp.dot(qp_ref[...].reshape(NB * H, 2 * HD), wqg_ref[...],
                 preferred_element_type=jnp.float32).reshape(NB, H, GH)
    kg = jnp.dot(kp_ref[...].reshape(NB * KVH, 2 * HD), wkg_ref[...],
                 preferred_element_type=jnp.float32).reshape(NB, KVH, GH)
    bc = bc_ref[...][:, None, :]
    bs = bs_ref[...][:, None, :]
    qg = qg * bc + _rot(qg) * bs
    kg = kg * bc + _rot(kg) * bs
    row = jax.lax.broadcasted_iota(jnp.int32, (NB, NB), 0)
    col = jax.lax.broadcasted_iota(jnp.int32, (NB, NB), 1)
    tri = row >= col
    e = e_ref[...]
    for h in range(H):
        L = jax.lax.dot_general(qg[:, h, :], kg[:, h // GROUP, :],
                                (((1,), (1,)), ((), ())),
                                preferred_element_type=jnp.float32)
        L = L / jnp.sqrt(jnp.float32(GH))
        L = jnp.where(tri, L, NEG)
        m = jnp.max(L, axis=-1, keepdims=True)
        p = jnp.exp(L - m)
        p = p / jnp.sum(p, axis=-1, keepdims=True)
        sel = ((p >= THRESH) | (row == col)) & tri
        key_exp = jax.lax.dot_general(e, sel.astype(jnp.float32).T,
                                      (((0,), (0,)), ((), ())),
                                      preferred_element_type=jnp.float32)
        addm_ref[h] = ((key_exp - 1.0) * 1e30).astype(jnp.bfloat16)


def _attn_kernel(q_ref, k_ref, v_ref, addm_ref, o_ref):
    i4 = pl.program_id(1)
    q = q_ref[0]
    e_sel = (jax.lax.broadcasted_iota(jnp.int32, (NB, RB), 0) ==
             i4 * (RB // BLK) +
             jax.lax.broadcasted_iota(jnp.int32, (NB, RB), 1) // BLK
             ).astype(jnp.bfloat16)
    keysr = jax.lax.broadcasted_iota(jnp.int32, (TK, RB), 0)
    qcols = jax.lax.broadcasted_iota(jnp.int32, (TK, RB), 1) + i4 * RB

    def score(j):
        kj = k_ref[0, pl.ds(j * TK, TK), :]
        sc = jax.lax.dot_general(kj, q, (((1,), (1,)), ((), ())),
                                 preferred_element_type=jnp.float32)
        am = addm_ref[0, pl.ds(j * TK, TK), :]
        madd = jax.lax.dot_general(am, e_sel, (((1,), (0,)), ((), ())),
                                   preferred_element_type=jnp.float32)
        return sc + madd

    nkt = RB // TK
    ntf = i4 * nkt

    def max_body(j, m):
        return jnp.maximum(m, jnp.max(score(j), axis=0, keepdims=True))

    m = jax.lax.fori_loop(0, ntf, max_body,
                          jnp.full((1, RB), NEG, jnp.float32))
    for t in range(nkt):
        j = ntf + t
        scd = jnp.where(keysr + j * TK <= qcols, score(j), NEG)
        m = jnp.maximum(m, jnp.max(scd, axis=0, keepdims=True))

    def acc_body(j, carry, sc):
        l, acc = carry
        p = jnp.exp2(sc - m)
        l = l + jnp.sum(p, axis=0, keepdims=True)
        vj = v_ref[0, pl.ds(j * TK, TK), :]
        acc = acc + jax.lax.dot_general(vj, p.astype(jnp.bfloat16),
                                        (((0,), (0,)), ((), ())),
                                        preferred_element_type=jnp.float32)
        return l, acc

    l0 = jnp.zeros((1, RB), jnp.float32)
    a0 = jnp.zeros((HD, RB), jnp.float32)
    l, acc = jax.lax.fori_loop(0, ntf,
                               lambda j, c: acc_body(j, c, score(j)),
                               (l0, a0))
    for t in range(nkt):
        j = ntf + t
        scd = jnp.where(keysr + j * TK <= qcols, score(j), NEG)
        l, acc = acc_body(j, (l, acc), scd)
    o_ref[0] = (acc / l).astype(jnp.bfloat16)


def _oproj_kernel(o_ref, w_ref, out_ref):
    x = o_ref[...].reshape(H * HD, RB)
    out_ref[...] = jax.lax.dot_general(x, w_ref[...], (((0,), (0,)), ((), ())),
                                       preferred_element_type=jnp.float32)


def kernel(hidden_states, cos, sin, block_cos, block_sin, Wq, bq, Wk, bk, Wv, bv, Wo, Wqg, Wkg):
    x = hidden_states.reshape(S, D)
    Wqkv = jnp.concatenate([Wq, Wk, Wv], axis=1)
    bqkv = jnp.concatenate([bq, bk, bv])[None, :]

    q, k, v, qp, kp = pl.pallas_call(
        _proj_kernel,
        grid=(NRB,),
        in_specs=[
            pl.BlockSpec((RB, D), lambda i: (i, 0)),
            pl.BlockSpec((D, (H + 2 * KVH) * HD), lambda i: (0, 0)),
            pl.BlockSpec((1, (H + 2 * KVH) * HD), lambda i: (0, 0)),
            pl.BlockSpec((RB, HD), lambda i: (i, 0)),
            pl.BlockSpec((RB, HD), lambda i: (i, 0)),
        ],
        out_specs=[
            pl.BlockSpec((H, RB, HD), lambda i: (0, i, 0)),
            pl.BlockSpec((KVH, RB, HD), lambda i: (0, i, 0)),
            pl.BlockSpec((KVH, RB, HD), lambda i: (0, i, 0)),
            pl.BlockSpec((RB // BLK, H, 2 * HD), lambda i: (i, 0, 0)),
            pl.BlockSpec((RB // BLK, KVH, 2 * HD), lambda i: (i, 0, 0)),
        ],
        out_shape=[
            jax.ShapeDtypeStruct((H, S, HD), jnp.bfloat16),
            jax.ShapeDtypeStruct((KVH, S, HD), jnp.bfloat16),
            jax.ShapeDtypeStruct((KVH, S, HD), jnp.bfloat16),
            jax.ShapeDtypeStruct((NB, H, 2 * HD), jnp.float32),
            jax.ShapeDtypeStruct((NB, KVH, 2 * HD), jnp.float32),
        ],
    )(x, Wqkv, bqkv, cos, sin)

    blk_cols = (jax.lax.broadcasted_iota(jnp.int32, (NB, S), 1) // BLK ==
                jax.lax.broadcasted_iota(jnp.int32, (NB, S), 0)).astype(jnp.float32)
    addm = pl.pallas_call(
        _gate_kernel,
        out_shape=jax.ShapeDtypeStruct((H, S, NB), jnp.bfloat16),
    )(qp, kp, Wqg, Wkg, block_cos, block_sin, blk_cols)

    oT = pl.pallas_call(
        _attn_kernel,
        grid=(H, NRB),
        in_specs=[
            pl.BlockSpec((1, RB, HD), lambda h, i: (h, i, 0)),
            pl.BlockSpec((1, S, HD), lambda h, i: (h // GROUP, 0, 0)),
            pl.BlockSpec((1, S, HD), lambda h, i: (h // GROUP, 0, 0)),
            pl.BlockSpec((1, S, NB), lambda h, i: (h, 0, 0)),
        ],
        out_specs=pl.BlockSpec((1, HD, RB), lambda h, i: (h, 0, i)),
        out_shape=jax.ShapeDtypeStruct((H, HD, S), jnp.bfloat16),
        compiler_params=pltpu.CompilerParams(
            dimension_semantics=("arbitrary", "arbitrary"),
        ),
    )(q, k, v, addm)

    Wo_bf = Wo.astype(jnp.bfloat16)
    out = pl.pallas_call(
        _oproj_kernel,
        grid=(NRB,),
        in_specs=[
            pl.BlockSpec((H, HD, RB), lambda i: (0, 0, i)),
            pl.BlockSpec((H * HD, D), lambda i: (0, 0)),
        ],
        out_specs=pl.BlockSpec((RB, D), lambda i: (i, 0)),
        out_shape=jax.ShapeDtypeStruct((S, D), jnp.float32),
    )(oT, Wo_bf)
    return out.reshape(1, S, D)


# speedup vs baseline: 1.3935x; 1.3935x over previous
import jax
import jax.numpy as jnp
from jax.experimental import pallas as pl
from jax.experimental.pallas import tpu as pltpu

S = 2048
D = 2048
H = 16
KVH = 4
HD = 128
BLK = 64
NB = S // BLK
GH = 128
GROUP = H // KVH
RB = 256
NRB = S // RB
TK = 256
SCALE = HD ** -0.5
LOG2E = 1.4426950408889634
QSCALE = SCALE * LOG2E
THRESH = 1.0 / NB
NEG = -1e30


def _rot(x):
    h = x.shape[-1] // 2
    return jnp.concatenate([-x[..., h:], x[..., :h]], axis=-1)


def _proj_kernel(x_ref, w_ref, b_ref, cos_ref, sin_ref,
                 q_ref, k_ref, v_ref, qp_ref, kp_ref):
    x = x_ref[...]
    qkv = jnp.dot(x, w_ref[...], preferred_element_type=jnp.float32) + b_ref[...]
    c = cos_ref[...]
    s = sin_ref[...]
    for h in range(H):
        qh = qkv[:, h * HD:(h + 1) * HD]
        q4 = qh.reshape(RB // BLK, BLK, HD)
        qp_ref[:, h, :HD] = jnp.mean(q4, axis=1)
        qp_ref[:, h, HD:] = jnp.max(q4, axis=1)
        q_ref[h] = ((qh * c + _rot(qh) * s) * QSCALE).astype(jnp.bfloat16)
    for g in range(KVH):
        kh = qkv[:, H * HD + g * HD: H * HD + (g + 1) * HD]
        k4 = kh.reshape(RB // BLK, BLK, HD)
        kp_ref[:, g, :HD] = jnp.mean(k4, axis=1)
        kp_ref[:, g, HD:] = jnp.max(k4, axis=1)
        k_ref[g] = (kh * c + _rot(kh) * s).astype(jnp.bfloat16)
        v_ref[g] = qkv[:, (H + KVH) * HD + g * HD:
                       (H + KVH) * HD + (g + 1) * HD].astype(jnp.bfloat16)


def _gate_kernel(qp_ref, kp_ref, wqg_ref, wkg_ref, bc_ref, bs_ref, e_ref,
                 addm_ref):
    qg = jnp.dot(qp_ref[...].reshape(NB * H, 2 * HD), wqg_ref[...],
                 preferred_element_type=jnp.float32).reshape(NB, H, GH)
    kg = jnp.dot(kp_ref[...].reshape(NB * KVH, 2 * HD), wkg_ref[...],
                 preferred_element_type=jnp.float32).reshape(NB, KVH, GH)
    bc = bc_ref[...][:, None, :]
    bs = bs_ref[...][:, None, :]
    qg = qg * bc + _rot(qg) * bs
    kg = kg * bc + _rot(kg) * bs
    row = jax.lax.broadcasted_iota(jnp.int32, (NB, NB), 0)
    col = jax.lax.broadcasted_iota(jnp.int32, (NB, NB), 1)
    tri = row >= col
    e = e_ref[...]
    for h in range(H):
        L = jax.lax.dot_general(qg[:, h, :], kg[:, h // GROUP, :],
                                (((1,), (1,)), ((), ())),
                                preferred_element_type=jnp.float32)
        L = L / jnp.sqrt(jnp.float32(GH))
        L = jnp.where(tri, L, NEG)
        m = jnp.max(L, axis=-1, keepdims=True)
        p = jnp.exp(L - m)
        p = p / jnp.sum(p, axis=-1, keepdims=True)
        sel = ((p >= THRESH) | (row == col)) & tri
        key_exp = jnp.dot(sel.astype(jnp.float32), e,
                          preferred_element_type=jnp.float32)
        addm_ref[h] = ((key_exp - 1.0) * 1e30).astype(jnp.bfloat16)


def _attn_kernel(q_ref, k_ref, v_ref, addm_ref, o_ref):
    i4 = pl.program_id(1)
    q = q_ref[0]
    e_sel = (jax.lax.broadcasted_iota(jnp.int32, (NB, RB), 0) ==
             i4 * (RB // BLK) +
             jax.lax.broadcasted_iota(jnp.int32, (NB, RB), 1) // BLK
             ).astype(jnp.bfloat16)
    keysr = jax.lax.broadcasted_iota(jnp.int32, (TK, RB), 0)
    qcols = jax.lax.broadcasted_iota(jnp.int32, (TK, RB), 1) + i4 * RB

    def score(j):
        kj = k_ref[0, pl.ds(j * TK, TK), :]
        sc = jax.lax.dot_general(kj, q, (((1,), (1,)), ((), ())),
                                 preferred_element_type=jnp.float32)
        am = addm_ref[0, :, pl.ds(j * TK, TK)]
        madd = jax.lax.dot_general(am, e_sel, (((0,), (0,)), ((), ())),
                                   preferred_element_type=jnp.float32)
        return sc + madd

    def process(m, l, acc, sc, j):
        mj = jnp.max(sc, axis=0, keepdims=True)
        m_new = jnp.maximum(m, mj)
        alpha = jnp.exp2(m - m_new)
        p = jnp.exp2(sc - m_new)
        l = l * alpha + jnp.sum(p, axis=0, keepdims=True)
        vj = v_ref[0, pl.ds(j * TK, TK), :]
        pv = jax.lax.dot_general(vj, p.astype(jnp.bfloat16),
                                 (((0,), (0,)), ((), ())),
                                 preferred_element_type=jnp.float32)
        acc = acc * alpha + pv
        return m_new, l, acc

    m0 = jnp.full((1, RB), NEG, jnp.float32)
    l0 = jnp.zeros((1, RB), jnp.float32)
    a0 = jnp.zeros((HD, RB), jnp.float32)

    def body(j, carry):
        m, l, acc, sc = carry
        sc_next = score(j)
        m, l, acc = process(m, l, acc, sc, j - 1)
        return m, l, acc, sc_next

    m, l, acc, sc = jax.lax.fori_loop(1, i4 + 1, body, (m0, l0, a0, score(0)))
    sc = jnp.where(keysr + i4 * TK <= qcols, sc, NEG)
    m, l, acc = process(m, l, acc, sc, i4)
    o_ref[0] = (acc / l).astype(jnp.bfloat16)


def _oproj_kernel(o_ref, w_ref, out_ref):
    x = o_ref[...].reshape(H * HD, RB)
    out_ref[...] = jax.lax.dot_general(x, w_ref[...], (((0,), (0,)), ((), ())),
                                       preferred_element_type=jnp.float32)


def kernel(hidden_states, cos, sin, block_cos, block_sin, Wq, bq, Wk, bk, Wv, bv, Wo, Wqg, Wkg):
    x = hidden_states.reshape(S, D)
    Wqkv = jnp.concatenate([Wq, Wk, Wv], axis=1)
    bqkv = jnp.concatenate([bq, bk, bv])[None, :]

    q, k, v, qp, kp = pl.pallas_call(
        _proj_kernel,
        grid=(NRB,),
        in_specs=[
            pl.BlockSpec((RB, D), lambda i: (i, 0)),
            pl.BlockSpec((D, (H + 2 * KVH) * HD), lambda i: (0, 0)),
            pl.BlockSpec((1, (H + 2 * KVH) * HD), lambda i: (0, 0)),
            pl.BlockSpec((RB, HD), lambda i: (i, 0)),
            pl.BlockSpec((RB, HD), lambda i: (i, 0)),
        ],
        out_specs=[
            pl.BlockSpec((H, RB, HD), lambda i: (0, i, 0)),
            pl.BlockSpec((KVH, RB, HD), lambda i: (0, i, 0)),
            pl.BlockSpec((KVH, RB, HD), lambda i: (0, i, 0)),
            pl.BlockSpec((RB // BLK, H, 2 * HD), lambda i: (i, 0, 0)),
            pl.BlockSpec((RB // BLK, KVH, 2 * HD), lambda i: (i, 0, 0)),
        ],
        out_shape=[
            jax.ShapeDtypeStruct((H, S, HD), jnp.bfloat16),
            jax.ShapeDtypeStruct((KVH, S, HD), jnp.bfloat16),
            jax.ShapeDtypeStruct((KVH, S, HD), jnp.bfloat16),
            jax.ShapeDtypeStruct((NB, H, 2 * HD), jnp.float32),
            jax.ShapeDtypeStruct((NB, KVH, 2 * HD), jnp.float32),
        ],
    )(x, Wqkv, bqkv, cos, sin)

    blk_cols = (jax.lax.broadcasted_iota(jnp.int32, (NB, S), 1) // BLK ==
                jax.lax.broadcasted_iota(jnp.int32, (NB, S), 0)).astype(jnp.float32)
    addm = pl.pallas_call(
        _gate_kernel,
        out_shape=jax.ShapeDtypeStruct((H, NB, S), jnp.bfloat16),
    )(qp, kp, Wqg, Wkg, block_cos, block_sin, blk_cols)

    oT = pl.pallas_call(
        _attn_kernel,
        grid=(H, NRB),
        in_specs=[
            pl.BlockSpec((1, RB, HD), lambda h, i: (h, i, 0)),
            pl.BlockSpec((1, S, HD), lambda h, i: (h // GROUP, 0, 0)),
            pl.BlockSpec((1, S, HD), lambda h, i: (h // GROUP, 0, 0)),
            pl.BlockSpec((1, NB, S), lambda h, i: (h, 0, 0)),
        ],
        out_specs=pl.BlockSpec((1, HD, RB), lambda h, i: (h, 0, i)),
        out_shape=jax.ShapeDtypeStruct((H, HD, S), jnp.bfloat16),
        compiler_params=pltpu.CompilerParams(
            dimension_semantics=("arbitrary", "arbitrary"),
        ),
    )(q, k, v, addm)

    Wo_bf = Wo.astype(jnp.bfloat16)
    out = pl.pallas_call(
        _oproj_kernel,
        grid=(NRB,),
        in_specs=[
            pl.BlockSpec((H, HD, RB), lambda i: (0, 0, i)),
            pl.BlockSpec((H * HD, D), lambda i: (0, 0)),
        ],
        out_specs=pl.BlockSpec((RB, D), lambda i: (i, 0)),
        out_shape=jax.ShapeDtypeStruct((S, D), jnp.float32),
    )(oT, Wo_bf)
    return out.reshape(1, S, D)


# speedup vs baseline: 1.4570x; 1.0456x over previous
import jax
import jax.numpy as jnp
from jax.experimental import pallas as pl
from jax.experimental.pallas import tpu as pltpu

S = 2048
D = 2048
H = 16
KVH = 4
HD = 128
BLK = 64
NB = S // BLK
GH = 128
GROUP = H // KVH
RB = 256
NRB = S // RB
TK = 256
SCALE = HD ** -0.5
LOG2E = 1.4426950408889634
QSCALE = SCALE * LOG2E
THRESH = 1.0 / NB
NEG = -1e30


def _rot(x):
    h = x.shape[-1] // 2
    return jnp.concatenate([-x[..., h:], x[..., :h]], axis=-1)


def _proj_kernel(x_ref, w_ref, b_ref, cos_ref, sin_ref,
                 q_ref, k_ref, v_ref, qp_ref, kp_ref):
    x = x_ref[...]
    qkv = jnp.dot(x, w_ref[...], preferred_element_type=jnp.float32) + b_ref[...]
    c = cos_ref[...]
    s = sin_ref[...]
    for h in range(H):
        qh = qkv[:, h * HD:(h + 1) * HD]
        q4 = qh.reshape(RB // BLK, BLK, HD)
        qp_ref[:, h, :HD] = jnp.mean(q4, axis=1)
        qp_ref[:, h, HD:] = jnp.max(q4, axis=1)
        q_ref[h] = ((qh * c + _rot(qh) * s) * QSCALE).astype(jnp.bfloat16)
    for g in range(KVH):
        kh = qkv[:, H * HD + g * HD: H * HD + (g + 1) * HD]
        k4 = kh.reshape(RB // BLK, BLK, HD)
        kp_ref[:, g, :HD] = jnp.mean(k4, axis=1)
        kp_ref[:, g, HD:] = jnp.max(k4, axis=1)
        k_ref[g] = (kh * c + _rot(kh) * s).astype(jnp.bfloat16)
        v_ref[g] = qkv[:, (H + KVH) * HD + g * HD:
                       (H + KVH) * HD + (g + 1) * HD].astype(jnp.bfloat16)


def _gate_kernel(qp_ref, kp_ref, wqg_ref, wkg_ref, bc_ref, bs_ref, e_ref,
                 addm_ref):
    qg = jnp.dot(qp_ref[...].reshape(NB * H, 2 * HD), wqg_ref[...],
                 preferred_element_type=jnp.float32).reshape(NB, H, GH)
    kg = jnp.dot(kp_ref[...].reshape(NB * KVH, 2 * HD), wkg_ref[...],
                 preferred_element_type=jnp.float32).reshape(NB, KVH, GH)
    bc = bc_ref[...][:, None, :]
    bs = bs_ref[...][:, None, :]
    qg = qg * bc + _rot(qg) * bs
    kg = kg * bc + _rot(kg) * bs
    row = jax.lax.broadcasted_iota(jnp.int32, (NB, NB), 0)
    col = jax.lax.broadcasted_iota(jnp.int32, (NB, NB), 1)
    tri = row >= col
    e = e_ref[...]
    for h in range(H):
        L = jax.lax.dot_general(qg[:, h, :], kg[:, h // GROUP, :],
                                (((1,), (1,)), ((), ())),
                                preferred_element_type=jnp.float32)
        L = L / jnp.sqrt(jnp.float32(GH))
        L = jnp.where(tri, L, NEG)
        m = jnp.max(L, axis=-1, keepdims=True)
        p = jnp.exp(L - m)
        p = p / jnp.sum(p, axis=-1, keepdims=True)
        sel = ((p >= THRESH) | (row == col)) & tri
        key_exp = jax.lax.dot_general(e, sel.astype(jnp.float32).T,
                                      (((0,), (0,)), ((), ())),
                                      preferred_element_type=jnp.float32)
        addm_ref[h] = ((key_exp - 1.0) * 1e30).astype(jnp.bfloat16)


def _attn_kernel(q_ref, k_ref, v_ref, addm_ref, o_ref):
    i4 = pl.program_id(1)
    q = q_ref[0]
    e_sel = (jax.lax.broadcasted_iota(jnp.int32, (NB, RB), 0) ==
             i4 * (RB // BLK) +
             jax.lax.broadcasted_iota(jnp.int32, (NB, RB), 1) // BLK
             ).astype(jnp.bfloat16)
    keysr = jax.lax.broadcasted_iota(jnp.int32, (TK, RB), 0)
    qcols = jax.lax.broadcasted_iota(jnp.int32, (TK, RB), 1) + i4 * RB

    def score(j):
        kj = k_ref[0, pl.ds(j * TK, TK), :]
        sc = jax.lax.dot_general(kj, q, (((1,), (1,)), ((), ())),
                                 preferred_element_type=jnp.float32)
        am = addm_ref[0, pl.ds(j * TK, TK), :]
        madd = jax.lax.dot_general(am, e_sel, (((1,), (0,)), ((), ())),
                                   preferred_element_type=jnp.float32)
        return sc + madd

    def process(m, l, acc, sc, j):
        mj = jnp.max(sc, axis=0, keepdims=True)
        m_new = jnp.maximum(m, mj)
        alpha = jnp.exp2(m - m_new)
        p = jnp.exp2(sc - m_new)
        l = l * alpha + jnp.sum(p, axis=0, keepdims=True)
        vj = v_ref[0, pl.ds(j * TK, TK), :]
        pv = jax.lax.dot_general(vj, p.astype(jnp.bfloat16),
                                 (((0,), (0,)), ((), ())),
                                 preferred_element_type=jnp.float32)
        acc = acc * alpha + pv
        return m_new, l, acc

    m0 = jnp.full((1, RB), NEG, jnp.float32)
    l0 = jnp.zeros((1, RB), jnp.float32)
    a0 = jnp.zeros((HD, RB), jnp.float32)

    def body(j, carry):
        m, l, acc, sc = carry
        sc_next = score(j)
        m, l, acc = process(m, l, acc, sc, j - 1)
        return m, l, acc, sc_next

    m, l, acc, sc = jax.lax.fori_loop(1, i4 + 1, body, (m0, l0, a0, score(0)))
    sc = jnp.where(keysr + i4 * TK <= qcols, sc, NEG)
    m, l, acc = process(m, l, acc, sc, i4)
    o_ref[0] = (acc / l).astype(jnp.bfloat16)


def _oproj_kernel(o_ref, w_ref, out_ref):
    x = o_ref[...].reshape(H * HD, RB)
    out_ref[...] = jax.lax.dot_general(x, w_ref[...], (((0,), (0,)), ((), ())),
                                       preferred_element_type=jnp.float32)


def kernel(hidden_states, cos, sin, block_cos, block_sin, Wq, bq, Wk, bk, Wv, bv, Wo, Wqg, Wkg):
    x = hidden_states.reshape(S, D)
    Wqkv = jnp.concatenate([Wq, Wk, Wv], axis=1)
    bqkv = jnp.concatenate([bq, bk, bv])[None, :]

    q, k, v, qp, kp = pl.pallas_call(
        _proj_kernel,
        grid=(NRB,),
        in_specs=[
            pl.BlockSpec((RB, D), lambda i: (i, 0)),
            pl.BlockSpec((D, (H + 2 * KVH) * HD), lambda i: (0, 0)),
            pl.BlockSpec((1, (H + 2 * KVH) * HD), lambda i: (0, 0)),
            pl.BlockSpec((RB, HD), lambda i: (i, 0)),
            pl.BlockSpec((RB, HD), lambda i: (i, 0)),
        ],
        out_specs=[
            pl.BlockSpec((H, RB, HD), lambda i: (0, i, 0)),
            pl.BlockSpec((KVH, RB, HD), lambda i: (0, i, 0)),
            pl.BlockSpec((KVH, RB, HD), lambda i: (0, i, 0)),
            pl.BlockSpec((RB // BLK, H, 2 * HD), lambda i: (i, 0, 0)),
            pl.BlockSpec((RB // BLK, KVH, 2 * HD), lambda i: (i, 0, 0)),
        ],
        out_shape=[
            jax.ShapeDtypeStruct((H, S, HD), jnp.bfloat16),
            jax.ShapeDtypeStruct((KVH, S, HD), jnp.bfloat16),
            jax.ShapeDtypeStruct((KVH, S, HD), jnp.bfloat16),
            jax.ShapeDtypeStruct((NB, H, 2 * HD), jnp.float32),
            jax.ShapeDtypeStruct((NB, KVH, 2 * HD), jnp.float32),
        ],
    )(x, Wqkv, bqkv, cos, sin)

    blk_cols = (jax.lax.broadcasted_iota(jnp.int32, (NB, S), 1) // BLK ==
                jax.lax.broadcasted_iota(jnp.int32, (NB, S), 0)).astype(jnp.float32)
    addm = pl.pallas_call(
        _gate_kernel,
        out_shape=jax.ShapeDtypeStruct((H, S, NB), jnp.bfloat16),
    )(qp, kp, Wqg, Wkg, block_cos, block_sin, blk_cols)

    oT = pl.pallas_call(
        _attn_kernel,
        grid=(H, NRB),
        in_specs=[
            pl.BlockSpec((1, RB, HD), lambda h, i: (h, i, 0)),
            pl.BlockSpec((1, S, HD), lambda h, i: (h // GROUP, 0, 0)),
            pl.BlockSpec((1, S, HD), lambda h, i: (h // GROUP, 0, 0)),
            pl.BlockSpec((1, S, NB), lambda h, i: (h, 0, 0)),
        ],
        out_specs=pl.BlockSpec((1, HD, RB), lambda h, i: (h, 0, i)),
        out_shape=jax.ShapeDtypeStruct((H, HD, S), jnp.bfloat16),
        compiler_params=pltpu.CompilerParams(
            dimension_semantics=("arbitrary", "arbitrary"),
        ),
    )(q, k, v, addm)

    Wo_bf = Wo.astype(jnp.bfloat16)
    out = pl.pallas_call(
        _oproj_kernel,
        grid=(NRB,),
        in_specs=[
            pl.BlockSpec((H, HD, RB), lambda i: (0, 0, i)),
            pl.BlockSpec((H * HD, D), lambda i: (0, 0)),
        ],
        out_specs=pl.BlockSpec((RB, D), lambda i: (i, 0)),
        out_shape=jax.ShapeDtypeStruct((S, D), jnp.float32),
    )(oT, Wo_bf)
    return out.reshape(1, S, D)
